# baseline (speedup 1.0000x reference)
import functools

import jax
import jax.numpy as jnp
from jax import lax
from jax.experimental import pallas as pl
from jax.experimental.pallas import tpu as pltpu
from jax.experimental.pallas import tpu_sc as plsc

_CHUNK = 128
_NW = 32
_SLOTS = 64
_PW = 16
_C = 65536


@jax.jit
def _sc_gather(table, idx4d):
    _, n_pass, c_per_pass, chunk = idx4d.shape
    rows = _NW * n_pass * c_per_pass * chunk
    d = table.shape[1]
    r_per_pass = c_per_pass * chunk
    mesh = plsc.VectorSubcoreMesh(core_axis_name="c", subcore_axis_name="s")
    nc = 2

    @functools.partial(
        pl.kernel,
        mesh=mesh,
        out_type=jax.ShapeDtypeStruct((rows, d), jnp.float32),
        scratch_types=[
            pltpu.VMEM((n_pass, c_per_pass, chunk), jnp.int32),
            pltpu.VMEM((r_per_pass, d), jnp.float32),
            pltpu.SemaphoreType.DMA,
        ],
        compiler_params=pltpu.CompilerParams(use_tc_tiling_on_sc=False),
    )
    def gather_kernel(table_hbm, idx_hbm, out_hbm, idx_v, rows_v, sem):
        wid = lax.axis_index("s") * nc + lax.axis_index("c")
        pltpu.sync_copy(idx_hbm.at[wid], idx_v)
        for p in range(n_pass):
            copies = []
            for j in range(c_per_pass):
                copies.append(
                    pltpu.async_copy(
                        table_hbm.at[idx_v.at[p, j]],
                        rows_v.at[pl.ds(j * chunk, chunk)],
                        sem,
                    )
                )
            for c in copies:
                c.wait()
            pltpu.sync_copy(
                rows_v,
                out_hbm.at[pl.ds((wid * n_pass + p) * r_per_pass, r_per_pass)],
            )

    return gather_kernel(table, idx4d)


def _pack_words(lo, hi):
    u_lo = lax.bitcast_convert_type(lo.astype(jnp.bfloat16), jnp.uint16)
    u_hi = lax.bitcast_convert_type(hi.astype(jnp.bfloat16), jnp.uint16)
    word = u_lo.astype(jnp.uint32) | (u_hi.astype(jnp.uint32) << 16)
    return lax.bitcast_convert_type(word, jnp.float32)


def _repack_body(t_ref, o_ref):
    c = t_ref.shape[1]
    for j2 in range(c // 1024):
        for s in range(2):
            off = j2 * 1024 + s * 512
            lo = jnp.concatenate(
                [t_ref[0:16, off + 128 * k : off + 128 * (k + 1)] for k in range(4)],
                axis=0,
            )
            hi = jnp.concatenate(
                [t_ref[16:32, off + 128 * k : off + 128 * (k + 1)] for k in range(4)],
                axis=0,
            )
            packed = _pack_words(
                jnp.maximum(lo, 0.0), jnp.maximum(hi, 0.0)
            )
            o_ref[
                j2 * 128 : (j2 + 1) * 128, s * 64 : (s + 1) * 64
            ] = jnp.swapaxes(packed, 0, 1)


@jax.jit
def _tc_repack(tT):
    d, vocab = tT.shape
    n_i = -(-vocab // _C)
    return pl.pallas_call(
        _repack_body,
        grid=(n_i,),
        in_specs=[pl.BlockSpec((d, _C), lambda i: (0, i))],
        out_specs=pl.BlockSpec((_C // 8, 128), lambda i: (i, 0)),
        out_shape=jax.ShapeDtypeStruct((n_i * _C // 8, 128), jnp.float32),
    )(tT)


def _linear_body(f_ref, wl_ref, wh_ref, b_ref, o_ref):
    u = lax.bitcast_convert_type(f_ref[...], jnp.uint32)
    f_lo = lax.bitcast_convert_type(u << 16, jnp.float32)
    f_hi = lax.bitcast_convert_type(u & jnp.uint32(0xFFFF0000), jnp.float32)
    acc = b_ref[...]
    for s in range(8):
        acc = acc + jnp.dot(
            f_lo[:, s, :], wl_ref[s], preferred_element_type=jnp.float32
        )
        acc = acc + jnp.dot(
            f_hi[:, s, :], wh_ref[s], preferred_element_type=jnp.float32
        )
    o_ref[...] = acc


@jax.jit
def _tc_linear(feats, wl, wh, b):
    batch = feats.shape[0]
    t = wl.shape[2]
    bt = 512
    grid = (batch // bt,)
    return pl.pallas_call(
        _linear_body,
        grid=grid,
        in_specs=[
            pl.BlockSpec((bt, 8, 128), lambda i: (i, 0, 0)),
            pl.BlockSpec((8, 128, t), lambda i: (0, 0, 0)),
            pl.BlockSpec((8, 128, t), lambda i: (0, 0, 0)),
            pl.BlockSpec((1, t), lambda i: (0, 0)),
        ],
        out_specs=pl.BlockSpec((bt, t), lambda i: (i, 0)),
        out_shape=jax.ShapeDtypeStruct((batch, t), jnp.float32),
    )(feats, wl, wh, b.reshape(1, t))


def kernel(x, emb_table, W, b):
    batch, inp = x.shape
    _, d = emb_table.shape
    t = W.shape[0]
    xp = jnp.concatenate(
        [x, jnp.broadcast_to(x[:, :1], (batch, _SLOTS - inp))], axis=1
    )
    fx = ((xp >> 10) << 10) + ((xp & 127) << 3) + ((xp >> 7) & 7)
    rows = batch * _SLOTS
    n_pass = 4
    c_per_pass = rows // (_NW * n_pass * _CHUNK)
    idx4d = fx.reshape(_NW, n_pass, c_per_pass, _CHUNK)
    t2 = _tc_repack(jnp.swapaxes(emb_table, 0, 1))
    t_packed = jnp.reshape(t2, (t2.shape[0] * (128 // _PW), _PW))
    feats = _sc_gather(t_packed, idx4d).reshape(batch, 8, 128)
    wt3 = jnp.pad(W.T, ((0, (_SLOTS - inp) * d), (0, 0))).reshape(
        _SLOTS, 2, _PW, t
    )
    wl = wt3[:, 0].reshape(8, 128, t)
    wh = wt3[:, 1].reshape(8, 128, t)
    return _tc_linear(feats, wl, wh, b)


# speedup vs baseline: 1.1083x; 1.1083x over previous
import functools

import jax
import jax.numpy as jnp
from jax import lax
from jax.experimental import pallas as pl
from jax.experimental.pallas import tpu as pltpu
from jax.experimental.pallas import tpu_sc as plsc

_CHUNK = 128
_NW = 32
_SLOTS = 32
_PW = 16
_C = 65536


@jax.jit
def _sc_gather(table, idx4d):
    _, n_pass, c_per_pass, chunk = idx4d.shape
    rows = _NW * n_pass * c_per_pass * chunk
    d = table.shape[1]
    r_per_pass = c_per_pass * chunk
    mesh = plsc.VectorSubcoreMesh(core_axis_name="c", subcore_axis_name="s")
    nc = 2

    @functools.partial(
        pl.kernel,
        mesh=mesh,
        out_type=jax.ShapeDtypeStruct((rows, d), jnp.float32),
        scratch_types=[
            pltpu.VMEM((n_pass, c_per_pass, chunk), jnp.int32),
            pltpu.VMEM((r_per_pass, d), jnp.float32),
            pltpu.SemaphoreType.DMA,
        ],
        compiler_params=pltpu.CompilerParams(use_tc_tiling_on_sc=False),
    )
    def gather_kernel(table_hbm, idx_hbm, out_hbm, idx_v, rows_v, sem):
        wid = lax.axis_index("s") * nc + lax.axis_index("c")
        pltpu.sync_copy(idx_hbm.at[wid], idx_v)
        for p in range(n_pass):
            copies = []
            for j in range(c_per_pass):
                copies.append(
                    pltpu.async_copy(
                        table_hbm.at[idx_v.at[p, j]],
                        rows_v.at[pl.ds(j * chunk, chunk)],
                        sem,
                    )
                )
            for c in copies:
                c.wait()
            pltpu.sync_copy(
                rows_v,
                out_hbm.at[pl.ds((wid * n_pass + p) * r_per_pass, r_per_pass)],
            )

    return gather_kernel(table, idx4d)


def _pack_words(lo, hi):
    u_lo = lax.bitcast_convert_type(lo.astype(jnp.bfloat16), jnp.uint16)
    u_hi = lax.bitcast_convert_type(hi.astype(jnp.bfloat16), jnp.uint16)
    word = u_lo.astype(jnp.uint32) | (u_hi.astype(jnp.uint32) << 16)
    return lax.bitcast_convert_type(word, jnp.float32)


def _repack_body(t_ref, o_ref):
    c = t_ref.shape[1]
    for j2 in range(c // 1024):
        for s in range(2):
            off = j2 * 1024 + s * 512
            lo = jnp.concatenate(
                [t_ref[0:16, off + 128 * k : off + 128 * (k + 1)] for k in range(4)],
                axis=0,
            )
            hi = jnp.concatenate(
                [t_ref[16:32, off + 128 * k : off + 128 * (k + 1)] for k in range(4)],
                axis=0,
            )
            packed = _pack_words(
                jnp.maximum(lo, 0.0), jnp.maximum(hi, 0.0)
            )
            o_ref[
                j2 * 128 : (j2 + 1) * 128, s * 64 : (s + 1) * 64
            ] = jnp.swapaxes(packed, 0, 1)


@jax.jit
def _tc_repack(tT):
    d, vocab = tT.shape
    n_i = -(-vocab // _C)
    return pl.pallas_call(
        _repack_body,
        grid=(n_i,),
        in_specs=[pl.BlockSpec((d, _C), lambda i: (0, i))],
        out_specs=pl.BlockSpec((_C // 8, 128), lambda i: (i, 0)),
        out_shape=jax.ShapeDtypeStruct((n_i * _C // 8, 128), jnp.float32),
    )(tT)


def _linear_body(f_ref, wl_ref, wh_ref, b_ref, o_ref):
    u = lax.bitcast_convert_type(f_ref[...], jnp.uint32)
    f_lo = lax.bitcast_convert_type(u << 16, jnp.float32)
    f_hi = lax.bitcast_convert_type(u & jnp.uint32(0xFFFF0000), jnp.float32)

    def compute(s0):
        acc = b_ref[...]
        for s in range(4):
            acc = acc + jnp.dot(
                f_lo[:, s0 + s, :], wl_ref[s], preferred_element_type=jnp.float32
            )
            acc = acc + jnp.dot(
                f_hi[:, s0 + s, :], wh_ref[s], preferred_element_type=jnp.float32
            )
        o_ref[...] = acc

    h = pl.program_id(1)

    @pl.when(h == 0)
    def _first_half():
        compute(0)

    @pl.when(h == 1)
    def _second_half():
        compute(4)


@jax.jit
def _tc_linear(feats, wl, wh, b):
    n_rows = feats.shape[0]
    t = wl.shape[2]
    bt = 256
    grid = (n_rows // bt, 2)
    return pl.pallas_call(
        _linear_body,
        grid=grid,
        in_specs=[
            pl.BlockSpec((bt, 8, 128), lambda i, h: (i, 0, 0)),
            pl.BlockSpec((4, 128, t), lambda i, h: (0, 0, 0)),
            pl.BlockSpec((4, 128, t), lambda i, h: (0, 0, 0)),
            pl.BlockSpec((1, t), lambda i, h: (0, 0)),
        ],
        out_specs=pl.BlockSpec((bt, t), lambda i, h: (i + h * (n_rows // bt), 0)),
        out_shape=jax.ShapeDtypeStruct((2 * n_rows, t), jnp.float32),
    )(feats, wl, wh, b.reshape(1, t))


def kernel(x, emb_table, W, b):
    batch, inp = x.shape
    _, d = emb_table.shape
    t = W.shape[0]
    xp = jnp.concatenate(
        [x, jnp.broadcast_to(x[:, :1], (batch, _SLOTS - inp))], axis=1
    )
    fx = ((xp >> 10) << 10) + ((xp & 127) << 3) + ((xp >> 7) & 7)
    rows = batch * _SLOTS
    n_pass = 2
    c_per_pass = rows // (_NW * n_pass * _CHUNK)
    fxp = jnp.swapaxes(fx.reshape(2, batch // 2, _SLOTS), 0, 1)
    idx4d = fxp.reshape(_NW, n_pass, c_per_pass, _CHUNK)
    t2 = _tc_repack(jnp.swapaxes(emb_table, 0, 1))
    t_packed = jnp.reshape(t2, (t2.shape[0] * (128 // _PW), _PW))
    feats = _sc_gather(t_packed, idx4d).reshape(batch // 2, 8, 128)
    wt3 = jnp.pad(W.T, ((0, (_SLOTS - inp) * d), (0, 0))).reshape(
        _SLOTS, 2, _PW, t
    )
    wl = wt3[:, 0].reshape(4, 128, t)
    wh = wt3[:, 1].reshape(4, 128, t)
    return _tc_linear(feats, wl, wh, b)
